# baseline (speedup 1.0000x reference)
import functools
import jax
import jax.numpy as jnp
from jax import lax
from jax.experimental import pallas as pl
from jax.experimental.pallas import tpu as pltpu
from jax.experimental.pallas import tpu_sc as plsc

NC = 2
NS = 16
K = 2048
BLK = 1024
NG = 64


def _elu(v):
    return jnp.where(v > 0, v, jnp.exp(v) - 1.0)


def _ceil_to(x, m):
    return ((x + m - 1) // m) * m


_SC_PARAMS = pltpu.CompilerParams(use_tc_tiling_on_sc=False)
_MESH = plsc.VectorSubcoreMesh(core_axis_name="c", subcore_axis_name="s")


def _make_sc_deg(n_pad, e_pad):
    rpt = n_pad // NS
    nit = e_pad // 2 // (NS * K)
    eph = e_pad // 2

    @functools.partial(
        pl.kernel,
        out_type=jax.ShapeDtypeStruct((NC, n_pad), jnp.float32),
        mesh=_MESH,
        compiler_params=_SC_PARAMS,
        scratch_types=dict(
            didx=pltpu.VMEM((K,), jnp.int32),
            ones=pltpu.VMEM((K,), jnp.float32),
            zb=pltpu.VMEM((rpt,), jnp.float32),
            deg_sh=pltpu.VMEM_SHARED((n_pad,), jnp.float32),
        ),
    )
    def sc_deg(dst_hbm, ones_hbm, z1_hbm, deg_out, didx, ones, zb, deg_sh):
        c = lax.axis_index("c")
        s = lax.axis_index("s")
        r0 = s * rpt
        pltpu.sync_copy(z1_hbm, zb)
        pltpu.sync_copy(zb, deg_sh.at[pl.ds(r0, rpt)])
        pltpu.sync_copy(ones_hbm, ones)
        plsc.subcore_barrier()
        base = c * eph + s * (eph // NS)

        def body(i, carry):
            off = pl.multiple_of(base + i * K, K)
            pltpu.sync_copy(dst_hbm.at[pl.ds(off, K)], didx)
            pltpu.sync_copy(ones, deg_sh.at[didx], add=True)
            return carry

        lax.fori_loop(0, nit, body, 0)
        plsc.subcore_barrier()
        pltpu.sync_copy(deg_sh.at[pl.ds(r0, rpt)], zb)
        pltpu.sync_copy(zb, deg_out.at[c, pl.ds(r0, rpt)])

    return sc_deg


def _make_sc_scalar(n_pad, e_pad):
    rpt = n_pad // NS
    nit = e_pad // 2 // (NS * K)
    eph = e_pad // 2

    @functools.partial(
        pl.kernel,
        out_type=jax.ShapeDtypeStruct((NC, n_pad), jnp.float32),
        mesh=_MESH,
        compiler_params=_SC_PARAMS,
        scratch_types=dict(
            sidx=pltpu.VMEM((K,), jnp.int32),
            didx=pltpu.VMEM((K,), jnp.int32),
            vals=pltpu.VMEM((K,), jnp.float32),
            zb=pltpu.VMEM((rpt,), jnp.float32),
            t_sh=pltpu.VMEM_SHARED((n_pad,), jnp.float32),
            sem=pltpu.SemaphoreType.DMA,
        ),
    )
    def sc_scalar(xq_hbm, src_hbm, dst_hbm, z1_hbm, t_out,
                  sidx, didx, vals, zb, t_sh, sem):
        c = lax.axis_index("c")
        s = lax.axis_index("s")
        r0 = s * rpt
        pltpu.sync_copy(z1_hbm, zb)
        pltpu.sync_copy(zb, t_sh.at[pl.ds(r0, rpt)])
        plsc.subcore_barrier()
        base = c * eph + s * (eph // NS)

        def body(i, carry):
            off = pl.multiple_of(base + i * K, K)
            pltpu.sync_copy(src_hbm.at[pl.ds(off, K)], sidx)
            pltpu.sync_copy(dst_hbm.at[pl.ds(off, K)], didx)
            pltpu.async_copy(xq_hbm.at[sidx], vals, sem).wait()
            pltpu.sync_copy(vals, t_sh.at[didx], add=True)
            return carry

        lax.fori_loop(0, nit, body, 0)
        plsc.subcore_barrier()
        pltpu.sync_copy(t_sh.at[pl.ds(r0, rpt)], zb)
        pltpu.sync_copy(zb, t_out.at[c, pl.ds(r0, rpt)])

    return sc_scalar


def _make_sc_wide(n_pad, e_pad):
    kw = 640
    rpt = n_pad // NS
    nq = 16
    qrt = rpt // nq
    nit = e_pad // (NS * kw)
    ept = e_pad // NS

    @functools.partial(
        pl.kernel,
        out_type=jax.ShapeDtypeStruct((n_pad, 64), jnp.float32),
        mesh=_MESH,
        compiler_params=_SC_PARAMS,
        scratch_types=dict(
            sidx0=pltpu.VMEM((kw,), jnp.int32),
            didx0=pltpu.VMEM((kw,), jnp.int32),
            rows0=pltpu.VMEM((kw, 16), jnp.float32),
            sidx1=pltpu.VMEM((kw,), jnp.int32),
            didx1=pltpu.VMEM((kw,), jnp.int32),
            rows1=pltpu.VMEM((kw, 16), jnp.float32),
            bb=pltpu.VMEM((qrt, 16), jnp.float32),
            agg_sh=pltpu.VMEM_SHARED((n_pad, 16), jnp.float32),
            gs0=pltpu.SemaphoreType.DMA,
            gs1=pltpu.SemaphoreType.DMA,
            is0=pltpu.SemaphoreType.DMA,
            is1=pltpu.SemaphoreType.DMA,
        ),
    )
    def sc_wide(qv_hbm, sidx4_hbm, dst_hbm, z16_hbm, a_out,
                sidx0, didx0, rows0, sidx1, didx1, rows1, bb, agg_sh,
                gs0, gs1, is0, is1):
        c = lax.axis_index("c")
        s = lax.axis_index("s")
        r0 = s * rpt
        base = s * ept
        bufs = ((sidx0, didx0, rows0, gs0, is0),
                (sidx1, didx1, rows1, gs1, is1))
        for p in range(2):
            j = c + 2 * p
            pltpu.sync_copy(z16_hbm, bb)
            for q in range(nq):
                pltpu.sync_copy(bb, agg_sh.at[pl.ds(r0 + q * qrt, qrt), :])
            plsc.subcore_barrier()

            def fetch_idx(b, i):
                sidx, didx, rows, gs, isem = bufs[b]
                off = pl.multiple_of(base + i * kw, kw)
                joff = pl.multiple_of(j * e_pad + off, kw)
                pltpu.async_copy(sidx4_hbm.at[pl.ds(joff, kw)], sidx, isem)
                pltpu.async_copy(dst_hbm.at[pl.ds(off, kw)], didx, isem)

            def launch_gather(b, i):
                sidx, didx, rows, gs, isem = bufs[b]
                off = pl.multiple_of(base + i * kw, kw)
                joff = pl.multiple_of(j * e_pad + off, kw)
                pltpu.make_async_copy(sidx4_hbm.at[pl.ds(joff, kw)], sidx,
                                      isem).wait()
                pltpu.make_async_copy(dst_hbm.at[pl.ds(off, kw)], didx,
                                      isem).wait()
                pltpu.async_copy(qv_hbm.at[sidx], rows, gs)

            def step(b, i):
                sidx, didx, rows, gs, isem = bufs[b]
                pltpu.make_async_copy(qv_hbm.at[sidx], rows, gs).wait()
                pltpu.sync_copy(rows, agg_sh.at[didx], add=True)

                @pl.when(i + 2 < nit)
                def _():
                    fetch_idx(b, i + 2)

                @pl.when(i + 1 < nit)
                def _():
                    launch_gather(1 - b, i + 1)

            fetch_idx(0, 0)
            fetch_idx(1, 1)
            launch_gather(0, 0)

            def body(i2, carry):
                step(0, 2 * i2)
                step(1, 2 * i2 + 1)
                return carry

            lax.fori_loop(0, nit // 2, body, 0)
            plsc.subcore_barrier()
            for q in range(nq):
                rr = r0 + q * qrt
                pltpu.sync_copy(agg_sh.at[pl.ds(rr, qrt), :], bb)
                pltpu.sync_copy(bb, a_out.at[pl.ds(rr, qrt), pl.ds(j * 16, 16)])
            plsc.subcore_barrier()

    return sc_wide


def _make_tc_shift4(e_pad):
    rows = e_pad // 128
    rb = 256
    grid = rows // rb

    def body(src_ref, out_ref):
        v = src_ref[...] * 4
        out_ref[...] = v[None, :, :] + lax.broadcasted_iota(jnp.int32, (4, rb, 128), 0)

    return pl.pallas_call(
        body,
        grid=(grid,),
        in_specs=[pl.BlockSpec((rb, 128), lambda i: (i, 0))],
        out_specs=pl.BlockSpec((4, rb, 128), lambda i: (0, i, 0)),
        out_shape=jax.ShapeDtypeStruct((4, rows, 128), jnp.int32),
    )


def _make_tc_prep(n_pad):
    nblk = n_pad // BLK

    def body(degp_ref, x_ref, dinv_ref, xq_ref):
        deg = 1.0 + degp_ref[0, :] + degp_ref[1, :]
        dinv = lax.rsqrt(deg)
        dinv_ref[...] = dinv[:, None]
        xq_ref[...] = dinv[:, None] * x_ref[...]

    return pl.pallas_call(
        body,
        grid=(nblk,),
        in_specs=[
            pl.BlockSpec((2, BLK), lambda i: (0, i)),
            pl.BlockSpec((BLK, 1), lambda i: (i, 0)),
        ],
        out_specs=[
            pl.BlockSpec((BLK, 1), lambda i: (i, 0)),
            pl.BlockSpec((BLK, 1), lambda i: (i, 0)),
        ],
        out_shape=[
            jax.ShapeDtypeStruct((n_pad, 1), jnp.float32),
            jax.ShapeDtypeStruct((n_pad, 1), jnp.float32),
        ],
    )


def _make_tc_layer1(n_pad, d):
    nblk = n_pad // BLK

    def body(tp_ref, xq_ref, dinv_ref, w0_ref, b0_ref, w1_ref, q_ref):
        xq = xq_ref[...]
        dinv = dinv_ref[...]
        u = dinv * (xq + tp_ref[0, :][:, None] + tp_ref[1, :][:, None])
        h = _elu(u * w0_ref[...] + b0_ref[...])
        q_ref[...] = dinv * jnp.dot(h, w1_ref[...],
                                    preferred_element_type=jnp.float32)

    return pl.pallas_call(
        body,
        grid=(nblk,),
        in_specs=[
            pl.BlockSpec((2, BLK), lambda i: (0, i)),
            pl.BlockSpec((BLK, 1), lambda i: (i, 0)),
            pl.BlockSpec((BLK, 1), lambda i: (i, 0)),
            pl.BlockSpec((1, d), lambda i: (0, 0)),
            pl.BlockSpec((1, d), lambda i: (0, 0)),
            pl.BlockSpec((d, d), lambda i: (0, 0)),
        ],
        out_specs=pl.BlockSpec((BLK, d), lambda i: (i, 0)),
        out_shape=jax.ShapeDtypeStruct((n_pad, d), jnp.float32),
    )


def _make_tc_layer(n_pad, d):
    nblk = n_pad // BLK

    def body(a_ref, q_ref, dinv_ref, b_ref, w_ref, qn_ref):
        dinv = dinv_ref[...]
        h = _elu(dinv * (a_ref[...] + q_ref[...]) + b_ref[...])
        qn_ref[...] = dinv * jnp.dot(h, w_ref[...],
                                     preferred_element_type=jnp.float32)

    return pl.pallas_call(
        body,
        grid=(nblk,),
        in_specs=[
            pl.BlockSpec((BLK, d), lambda i: (i, 0)),
            pl.BlockSpec((BLK, d), lambda i: (i, 0)),
            pl.BlockSpec((BLK, 1), lambda i: (i, 0)),
            pl.BlockSpec((1, d), lambda i: (0, 0)),
            pl.BlockSpec((d, d), lambda i: (0, 0)),
        ],
        out_specs=pl.BlockSpec((BLK, d), lambda i: (i, 0)),
        out_shape=jax.ShapeDtypeStruct((n_pad, d), jnp.float32),
    )


def _make_tc_final(n_pad, d):
    nblk = n_pad // BLK

    def body(a_ref, q_ref, dinv_ref, b_ref, batch_ref, lw_ref, lb_ref,
             out_ref, sums, cnt):
        i = pl.program_id(0)
        dinv = dinv_ref[...]
        h = _elu(dinv * (a_ref[...] + q_ref[...]) + b_ref[...])
        gid = lax.broadcasted_iota(jnp.int32, (BLK, NG), 1)
        oh = (batch_ref[...] == gid).astype(jnp.float32)
        psum = lax.dot_general(oh, h, (((0,), (0,)), ((), ())),
                               preferred_element_type=jnp.float32)
        pcnt = jnp.sum(oh, axis=0)[:, None]

        @pl.when(i == 0)
        def _():
            sums[...] = psum
            cnt[...] = pcnt

        @pl.when(i > 0)
        def _():
            sums[...] += psum
            cnt[...] += pcnt

        @pl.when(i == nblk - 1)
        def _():
            pooled = sums[...] / jnp.maximum(cnt[...], 1.0)
            out_ref[...] = jnp.dot(pooled, lw_ref[...],
                                   preferred_element_type=jnp.float32) + lb_ref[...]

    return pl.pallas_call(
        body,
        grid=(nblk,),
        in_specs=[
            pl.BlockSpec((BLK, d), lambda i: (i, 0)),
            pl.BlockSpec((BLK, d), lambda i: (i, 0)),
            pl.BlockSpec((BLK, 1), lambda i: (i, 0)),
            pl.BlockSpec((1, d), lambda i: (0, 0)),
            pl.BlockSpec((BLK, 1), lambda i: (i, 0)),
            pl.BlockSpec((d, 1), lambda i: (0, 0)),
            pl.BlockSpec((1, 1), lambda i: (0, 0)),
        ],
        out_specs=pl.BlockSpec((NG, 1), lambda i: (0, 0)),
        out_shape=jax.ShapeDtypeStruct((NG, 1), jnp.float32),
        scratch_shapes=[
            pltpu.VMEM((NG, d), jnp.float32),
            pltpu.VMEM((NG, 1), jnp.float32),
        ],
    )


def kernel(x, edge_index, batch, W0, b0, W_rest, b_rest, lin_W, lin_b):
    n = x.shape[0]
    e = edge_index.shape[1]
    d = W0.shape[1]
    nl = W_rest.shape[0] + 1

    n_pad = _ceil_to(n + 1, BLK)
    e_pad = _ceil_to(e, 327680)
    dummy = n

    src = edge_index[0].astype(jnp.int32)
    dst = edge_index[1].astype(jnp.int32)
    src_p = jnp.pad(src, (0, e_pad - e))
    dst_p = jnp.pad(dst, (0, e_pad - e), constant_values=dummy)
    x_p = jnp.pad(x.astype(jnp.float32), ((0, n_pad - n), (0, 0)))
    batch_p = jnp.pad(batch.astype(jnp.int32), (0, n_pad - n),
                      constant_values=-1).reshape(n_pad, 1)

    ones_k = jnp.ones((K,), jnp.float32)
    z1 = jnp.zeros((n_pad // NS,), jnp.float32)
    z16 = jnp.zeros((n_pad // NS // 16, 16), jnp.float32)

    sidx4 = _make_tc_shift4(e_pad)(src_p.reshape(e_pad // 128, 128))
    sidx4_flat = sidx4.reshape(4 * e_pad)

    deg_p = _make_sc_deg(n_pad, e_pad)(dst_p, ones_k, z1)
    dinv, xq = _make_tc_prep(n_pad)(deg_p, x_p)

    t_p = _make_sc_scalar(n_pad, e_pad)(xq.reshape(n_pad), src_p, dst_p, z1)
    q = _make_tc_layer1(n_pad, d)(t_p, xq, dinv, W0, b0.reshape(1, d),
                                  W_rest[0])

    sc_wide = _make_sc_wide(n_pad, e_pad)
    tc_layer = _make_tc_layer(n_pad, d)
    for l in range(nl - 2):
        a = sc_wide(q.reshape(4 * n_pad, 16), sidx4_flat, dst_p, z16)
        q = tc_layer(a, q, dinv, b_rest[l].reshape(1, d), W_rest[l + 1])

    a = sc_wide(q.reshape(4 * n_pad, 16), sidx4_flat, dst_p, z16)
    out = _make_tc_final(n_pad, d)(a, q, dinv, b_rest[nl - 2].reshape(1, d),
                                   batch_p, lin_W, lin_b.reshape(1, 1))
    return out.reshape(NG)


# speedup vs baseline: 1.0995x; 1.0995x over previous
import functools
import jax
import jax.numpy as jnp
from jax import lax
from jax.experimental import pallas as pl
from jax.experimental.pallas import tpu as pltpu
from jax.experimental.pallas import tpu_sc as plsc

NC = 2
NS = 16
K = 2048
BLK = 1024
NG = 64


def _elu(v):
    return jnp.where(v > 0, v, jnp.exp(v) - 1.0)


def _ceil_to(x, m):
    return ((x + m - 1) // m) * m


_SC_PARAMS = pltpu.CompilerParams(use_tc_tiling_on_sc=False)
_MESH = plsc.VectorSubcoreMesh(core_axis_name="c", subcore_axis_name="s")


def _make_sc_deg(n_pad, e_pad):
    rpt = n_pad // NS
    nit = e_pad // 2 // (NS * K)
    eph = e_pad // 2

    @functools.partial(
        pl.kernel,
        out_type=jax.ShapeDtypeStruct((NC, n_pad), jnp.float32),
        mesh=_MESH,
        compiler_params=_SC_PARAMS,
        scratch_types=dict(
            didx=pltpu.VMEM((K,), jnp.int32),
            ones=pltpu.VMEM((K,), jnp.float32),
            zb=pltpu.VMEM((rpt,), jnp.float32),
            deg_sh=pltpu.VMEM_SHARED((n_pad,), jnp.float32),
        ),
    )
    def sc_deg(dst_hbm, ones_hbm, z1_hbm, deg_out, didx, ones, zb, deg_sh):
        c = lax.axis_index("c")
        s = lax.axis_index("s")
        r0 = s * rpt
        pltpu.sync_copy(z1_hbm, zb)
        pltpu.sync_copy(zb, deg_sh.at[pl.ds(r0, rpt)])
        pltpu.sync_copy(ones_hbm, ones)
        plsc.subcore_barrier()
        base = c * eph + s * (eph // NS)

        def body(i, carry):
            off = pl.multiple_of(base + i * K, K)
            pltpu.sync_copy(dst_hbm.at[pl.ds(off, K)], didx)
            pltpu.sync_copy(ones, deg_sh.at[didx], add=True)
            return carry

        lax.fori_loop(0, nit, body, 0)
        plsc.subcore_barrier()
        pltpu.sync_copy(deg_sh.at[pl.ds(r0, rpt)], zb)
        pltpu.sync_copy(zb, deg_out.at[c, pl.ds(r0, rpt)])

    return sc_deg


def _make_sc_scalar(n_pad, e_pad):
    rpt = n_pad // NS
    nit = e_pad // 2 // (NS * K)
    eph = e_pad // 2

    @functools.partial(
        pl.kernel,
        out_type=jax.ShapeDtypeStruct((NC, n_pad), jnp.float32),
        mesh=_MESH,
        compiler_params=_SC_PARAMS,
        scratch_types=dict(
            sidx=pltpu.VMEM((K,), jnp.int32),
            didx=pltpu.VMEM((K,), jnp.int32),
            vals=pltpu.VMEM((K,), jnp.float32),
            zb=pltpu.VMEM((rpt,), jnp.float32),
            t_sh=pltpu.VMEM_SHARED((n_pad,), jnp.float32),
            sem=pltpu.SemaphoreType.DMA,
        ),
    )
    def sc_scalar(xq_hbm, src_hbm, dst_hbm, z1_hbm, t_out,
                  sidx, didx, vals, zb, t_sh, sem):
        c = lax.axis_index("c")
        s = lax.axis_index("s")
        r0 = s * rpt
        pltpu.sync_copy(z1_hbm, zb)
        pltpu.sync_copy(zb, t_sh.at[pl.ds(r0, rpt)])
        plsc.subcore_barrier()
        base = c * eph + s * (eph // NS)

        def body(i, carry):
            off = pl.multiple_of(base + i * K, K)
            pltpu.sync_copy(src_hbm.at[pl.ds(off, K)], sidx)
            pltpu.sync_copy(dst_hbm.at[pl.ds(off, K)], didx)
            pltpu.async_copy(xq_hbm.at[sidx], vals, sem).wait()
            pltpu.sync_copy(vals, t_sh.at[didx], add=True)
            return carry

        lax.fori_loop(0, nit, body, 0)
        plsc.subcore_barrier()
        pltpu.sync_copy(t_sh.at[pl.ds(r0, rpt)], zb)
        pltpu.sync_copy(zb, t_out.at[c, pl.ds(r0, rpt)])

    return sc_scalar


def _make_sc_wide(n_pad, e_pad):
    kw = 640
    rpt = n_pad // NS
    nq = 16
    qrt = rpt // nq
    nit = e_pad // (NS * kw)
    ept = e_pad // NS

    @functools.partial(
        pl.kernel,
        out_type=jax.ShapeDtypeStruct((n_pad, 64), jnp.float32),
        mesh=_MESH,
        compiler_params=_SC_PARAMS,
        scratch_types=dict(
            sidx0=pltpu.VMEM((kw,), jnp.int32),
            didx0=pltpu.VMEM((kw,), jnp.int32),
            rows0=pltpu.VMEM((kw, 16), jnp.float32),
            sidx1=pltpu.VMEM((kw,), jnp.int32),
            didx1=pltpu.VMEM((kw,), jnp.int32),
            rows1=pltpu.VMEM((kw, 16), jnp.float32),
            bb=pltpu.VMEM((qrt, 16), jnp.float32),
            agg_sh=pltpu.VMEM_SHARED((n_pad, 16), jnp.float32),
            gs0=pltpu.SemaphoreType.DMA,
            gs1=pltpu.SemaphoreType.DMA,
            is0=pltpu.SemaphoreType.DMA,
            is1=pltpu.SemaphoreType.DMA,
        ),
    )
    def sc_wide(qv_hbm, sidx4_hbm, dst_hbm, z16_hbm, a_out,
                sidx0, didx0, rows0, sidx1, didx1, rows1, bb, agg_sh,
                gs0, gs1, is0, is1):
        c = lax.axis_index("c")
        s = lax.axis_index("s")
        r0 = s * rpt
        base = s * ept
        bufs = ((sidx0, didx0, rows0, gs0, is0),
                (sidx1, didx1, rows1, gs1, is1))
        for p in range(2):
            j = c + 2 * p
            pltpu.sync_copy(z16_hbm, bb)
            for q in range(nq):
                pltpu.sync_copy(bb, agg_sh.at[pl.ds(r0 + q * qrt, qrt), :])
            plsc.subcore_barrier()

            def fetch_idx(b, i):
                sidx, didx, rows, gs, isem = bufs[b]
                off = pl.multiple_of(base + i * kw, kw)
                joff = pl.multiple_of(j * e_pad + off, kw)
                pltpu.async_copy(sidx4_hbm.at[pl.ds(joff, kw)], sidx, isem)
                pltpu.async_copy(dst_hbm.at[pl.ds(off, kw)], didx, isem)

            def launch_gather(b, i):
                sidx, didx, rows, gs, isem = bufs[b]
                off = pl.multiple_of(base + i * kw, kw)
                joff = pl.multiple_of(j * e_pad + off, kw)
                pltpu.make_async_copy(sidx4_hbm.at[pl.ds(joff, kw)], sidx,
                                      isem).wait()
                pltpu.make_async_copy(dst_hbm.at[pl.ds(off, kw)], didx,
                                      isem).wait()
                pltpu.async_copy(qv_hbm.at[sidx], rows, gs)

            def step(b, i):
                sidx, didx, rows, gs, isem = bufs[b]
                pltpu.make_async_copy(qv_hbm.at[sidx], rows, gs).wait()
                pltpu.sync_copy(rows, agg_sh.at[didx], add=True)

                @pl.when(i + 2 < nit)
                def _():
                    fetch_idx(b, i + 2)
                    launch_gather(b, i + 2)

            fetch_idx(0, 0)
            fetch_idx(1, 1)
            launch_gather(0, 0)
            launch_gather(1, 1)

            def body(i2, carry):
                step(0, 2 * i2)
                step(1, 2 * i2 + 1)
                return carry

            lax.fori_loop(0, nit // 2, body, 0)
            plsc.subcore_barrier()
            for q in range(nq):
                rr = r0 + q * qrt
                pltpu.sync_copy(agg_sh.at[pl.ds(rr, qrt), :], bb)
                pltpu.sync_copy(bb, a_out.at[pl.ds(rr, qrt), pl.ds(j * 16, 16)])
            plsc.subcore_barrier()

    return sc_wide


def _make_tc_shift4(e_pad):
    rows = e_pad // 128
    rb = 256
    grid = rows // rb

    def body(src_ref, out_ref):
        v = src_ref[...] * 4
        out_ref[...] = v[None, :, :] + lax.broadcasted_iota(jnp.int32, (4, rb, 128), 0)

    return pl.pallas_call(
        body,
        grid=(grid,),
        in_specs=[pl.BlockSpec((rb, 128), lambda i: (i, 0))],
        out_specs=pl.BlockSpec((4, rb, 128), lambda i: (0, i, 0)),
        out_shape=jax.ShapeDtypeStruct((4, rows, 128), jnp.int32),
    )


def _make_tc_prep(n_pad):
    nblk = n_pad // BLK

    def body(degp_ref, x_ref, dinv_ref, xq_ref):
        deg = 1.0 + degp_ref[0, :] + degp_ref[1, :]
        dinv = lax.rsqrt(deg)
        dinv_ref[...] = dinv[:, None]
        xq_ref[...] = dinv[:, None] * x_ref[...]

    return pl.pallas_call(
        body,
        grid=(nblk,),
        in_specs=[
            pl.BlockSpec((2, BLK), lambda i: (0, i)),
            pl.BlockSpec((BLK, 1), lambda i: (i, 0)),
        ],
        out_specs=[
            pl.BlockSpec((BLK, 1), lambda i: (i, 0)),
            pl.BlockSpec((BLK, 1), lambda i: (i, 0)),
        ],
        out_shape=[
            jax.ShapeDtypeStruct((n_pad, 1), jnp.float32),
            jax.ShapeDtypeStruct((n_pad, 1), jnp.float32),
        ],
    )


def _make_tc_layer1(n_pad, d):
    nblk = n_pad // BLK

    def body(tp_ref, xq_ref, dinv_ref, w0_ref, b0_ref, w1_ref, q_ref):
        xq = xq_ref[...]
        dinv = dinv_ref[...]
        u = dinv * (xq + tp_ref[0, :][:, None] + tp_ref[1, :][:, None])
        h = _elu(u * w0_ref[...] + b0_ref[...])
        q_ref[...] = dinv * jnp.dot(h, w1_ref[...],
                                    preferred_element_type=jnp.float32)

    return pl.pallas_call(
        body,
        grid=(nblk,),
        in_specs=[
            pl.BlockSpec((2, BLK), lambda i: (0, i)),
            pl.BlockSpec((BLK, 1), lambda i: (i, 0)),
            pl.BlockSpec((BLK, 1), lambda i: (i, 0)),
            pl.BlockSpec((1, d), lambda i: (0, 0)),
            pl.BlockSpec((1, d), lambda i: (0, 0)),
            pl.BlockSpec((d, d), lambda i: (0, 0)),
        ],
        out_specs=pl.BlockSpec((BLK, d), lambda i: (i, 0)),
        out_shape=jax.ShapeDtypeStruct((n_pad, d), jnp.float32),
    )


def _make_tc_layer(n_pad, d):
    nblk = n_pad // BLK

    def body(a_ref, q_ref, dinv_ref, b_ref, w_ref, qn_ref):
        dinv = dinv_ref[...]
        h = _elu(dinv * (a_ref[...] + q_ref[...]) + b_ref[...])
        qn_ref[...] = dinv * jnp.dot(h, w_ref[...],
                                     preferred_element_type=jnp.float32)

    return pl.pallas_call(
        body,
        grid=(nblk,),
        in_specs=[
            pl.BlockSpec((BLK, d), lambda i: (i, 0)),
            pl.BlockSpec((BLK, d), lambda i: (i, 0)),
            pl.BlockSpec((BLK, 1), lambda i: (i, 0)),
            pl.BlockSpec((1, d), lambda i: (0, 0)),
            pl.BlockSpec((d, d), lambda i: (0, 0)),
        ],
        out_specs=pl.BlockSpec((BLK, d), lambda i: (i, 0)),
        out_shape=jax.ShapeDtypeStruct((n_pad, d), jnp.float32),
    )


def _make_tc_final(n_pad, d):
    nblk = n_pad // BLK

    def body(a_ref, q_ref, dinv_ref, b_ref, batch_ref, lw_ref, lb_ref,
             out_ref, sums, cnt):
        i = pl.program_id(0)
        dinv = dinv_ref[...]
        h = _elu(dinv * (a_ref[...] + q_ref[...]) + b_ref[...])
        gid = lax.broadcasted_iota(jnp.int32, (BLK, NG), 1)
        oh = (batch_ref[...] == gid).astype(jnp.float32)
        psum = lax.dot_general(oh, h, (((0,), (0,)), ((), ())),
                               preferred_element_type=jnp.float32)
        pcnt = jnp.sum(oh, axis=0)[:, None]

        @pl.when(i == 0)
        def _():
            sums[...] = psum
            cnt[...] = pcnt

        @pl.when(i > 0)
        def _():
            sums[...] += psum
            cnt[...] += pcnt

        @pl.when(i == nblk - 1)
        def _():
            pooled = sums[...] / jnp.maximum(cnt[...], 1.0)
            out_ref[...] = jnp.dot(pooled, lw_ref[...],
                                   preferred_element_type=jnp.float32) + lb_ref[...]

    return pl.pallas_call(
        body,
        grid=(nblk,),
        in_specs=[
            pl.BlockSpec((BLK, d), lambda i: (i, 0)),
            pl.BlockSpec((BLK, d), lambda i: (i, 0)),
            pl.BlockSpec((BLK, 1), lambda i: (i, 0)),
            pl.BlockSpec((1, d), lambda i: (0, 0)),
            pl.BlockSpec((BLK, 1), lambda i: (i, 0)),
            pl.BlockSpec((d, 1), lambda i: (0, 0)),
            pl.BlockSpec((1, 1), lambda i: (0, 0)),
        ],
        out_specs=pl.BlockSpec((NG, 1), lambda i: (0, 0)),
        out_shape=jax.ShapeDtypeStruct((NG, 1), jnp.float32),
        scratch_shapes=[
            pltpu.VMEM((NG, d), jnp.float32),
            pltpu.VMEM((NG, 1), jnp.float32),
        ],
    )


def kernel(x, edge_index, batch, W0, b0, W_rest, b_rest, lin_W, lin_b):
    n = x.shape[0]
    e = edge_index.shape[1]
    d = W0.shape[1]
    nl = W_rest.shape[0] + 1

    n_pad = _ceil_to(n + 1, BLK)
    e_pad = _ceil_to(e, 327680)
    dummy = n

    src = edge_index[0].astype(jnp.int32)
    dst = edge_index[1].astype(jnp.int32)
    src_p = jnp.pad(src, (0, e_pad - e))
    dst_p = jnp.pad(dst, (0, e_pad - e), constant_values=dummy)
    x_p = jnp.pad(x.astype(jnp.float32), ((0, n_pad - n), (0, 0)))
    batch_p = jnp.pad(batch.astype(jnp.int32), (0, n_pad - n),
                      constant_values=-1).reshape(n_pad, 1)

    ones_k = jnp.ones((K,), jnp.float32)
    z1 = jnp.zeros((n_pad // NS,), jnp.float32)
    z16 = jnp.zeros((n_pad // NS // 16, 16), jnp.float32)

    sidx4 = _make_tc_shift4(e_pad)(src_p.reshape(e_pad // 128, 128))
    sidx4_flat = sidx4.reshape(4 * e_pad)

    deg_p = _make_sc_deg(n_pad, e_pad)(dst_p, ones_k, z1)
    dinv, xq = _make_tc_prep(n_pad)(deg_p, x_p)

    t_p = _make_sc_scalar(n_pad, e_pad)(xq.reshape(n_pad), src_p, dst_p, z1)
    q = _make_tc_layer1(n_pad, d)(t_p, xq, dinv, W0, b0.reshape(1, d),
                                  W_rest[0])

    sc_wide = _make_sc_wide(n_pad, e_pad)
    tc_layer = _make_tc_layer(n_pad, d)
    for l in range(nl - 2):
        a = sc_wide(q.reshape(4 * n_pad, 16), sidx4_flat, dst_p, z16)
        q = tc_layer(a, q, dinv, b_rest[l].reshape(1, d), W_rest[l + 1])

    a = sc_wide(q.reshape(4 * n_pad, 16), sidx4_flat, dst_p, z16)
    out = _make_tc_final(n_pad, d)(a, q, dinv, b_rest[nl - 2].reshape(1, d),
                                   batch_p, lin_W, lin_b.reshape(1, 1))
    return out.reshape(NG)


# speedup vs baseline: 1.1102x; 1.0097x over previous
import functools
import jax
import jax.numpy as jnp
from jax import lax
from jax.experimental import pallas as pl
from jax.experimental.pallas import tpu as pltpu
from jax.experimental.pallas import tpu_sc as plsc

NC = 2
NS = 16
K = 2560
BLK = 1024
NG = 64


def _elu(v):
    return jnp.where(v > 0, v, jnp.exp(v) - 1.0)


def _ceil_to(x, m):
    return ((x + m - 1) // m) * m


_SC_PARAMS = pltpu.CompilerParams(use_tc_tiling_on_sc=False)
_MESH = plsc.VectorSubcoreMesh(core_axis_name="c", subcore_axis_name="s")


def _make_sc_deg(n_pad, e_pad):
    rpt = n_pad // NS
    nit = e_pad // 2 // (NS * K)
    eph = e_pad // 2

    @functools.partial(
        pl.kernel,
        out_type=jax.ShapeDtypeStruct((NC, n_pad), jnp.float32),
        mesh=_MESH,
        compiler_params=_SC_PARAMS,
        scratch_types=dict(
            didx0=pltpu.VMEM((K,), jnp.int32),
            didx1=pltpu.VMEM((K,), jnp.int32),
            ones=pltpu.VMEM((K,), jnp.float32),
            zb=pltpu.VMEM((rpt,), jnp.float32),
            deg_sh=pltpu.VMEM_SHARED((n_pad,), jnp.float32),
            is0=pltpu.SemaphoreType.DMA,
            is1=pltpu.SemaphoreType.DMA,
        ),
    )
    def sc_deg(dst_hbm, ones_hbm, z1_hbm, deg_out, didx0, didx1, ones, zb,
               deg_sh, is0, is1):
        c = lax.axis_index("c")
        s = lax.axis_index("s")
        r0 = s * rpt
        pltpu.sync_copy(z1_hbm, zb)
        pltpu.sync_copy(zb, deg_sh.at[pl.ds(r0, rpt)])
        pltpu.sync_copy(ones_hbm, ones)
        plsc.subcore_barrier()
        base = c * eph + s * (eph // NS)
        bufs = ((didx0, is0), (didx1, is1))

        def fetch(b, i):
            didx, isem = bufs[b]
            off = pl.multiple_of(base + i * K, K)
            pltpu.async_copy(dst_hbm.at[pl.ds(off, K)], didx, isem)

        def step(b, i):
            didx, isem = bufs[b]
            off = pl.multiple_of(base + i * K, K)
            pltpu.make_async_copy(dst_hbm.at[pl.ds(off, K)], didx, isem).wait()
            pltpu.sync_copy(ones, deg_sh.at[didx], add=True)

            @pl.when(i + 2 < nit)
            def _():
                fetch(b, i + 2)

        fetch(0, 0)
        fetch(1, 1)

        def body(i2, carry):
            step(0, 2 * i2)
            step(1, 2 * i2 + 1)
            return carry

        lax.fori_loop(0, nit // 2, body, 0)
        plsc.subcore_barrier()
        pltpu.sync_copy(deg_sh.at[pl.ds(r0, rpt)], zb)
        pltpu.sync_copy(zb, deg_out.at[c, pl.ds(r0, rpt)])

    return sc_deg


def _make_sc_scalar(n_pad, e_pad):
    rpt = n_pad // NS
    nit = e_pad // 2 // (NS * K)
    eph = e_pad // 2

    @functools.partial(
        pl.kernel,
        out_type=jax.ShapeDtypeStruct((NC, n_pad), jnp.float32),
        mesh=_MESH,
        compiler_params=_SC_PARAMS,
        scratch_types=dict(
            sidx0=pltpu.VMEM((K,), jnp.int32),
            didx0=pltpu.VMEM((K,), jnp.int32),
            vals0=pltpu.VMEM((K,), jnp.float32),
            sidx1=pltpu.VMEM((K,), jnp.int32),
            didx1=pltpu.VMEM((K,), jnp.int32),
            vals1=pltpu.VMEM((K,), jnp.float32),
            zb=pltpu.VMEM((rpt,), jnp.float32),
            t_sh=pltpu.VMEM_SHARED((n_pad,), jnp.float32),
            gs0=pltpu.SemaphoreType.DMA,
            gs1=pltpu.SemaphoreType.DMA,
            is0=pltpu.SemaphoreType.DMA,
            is1=pltpu.SemaphoreType.DMA,
        ),
    )
    def sc_scalar(xq_hbm, src_hbm, dst_hbm, z1_hbm, t_out,
                  sidx0, didx0, vals0, sidx1, didx1, vals1, zb, t_sh,
                  gs0, gs1, is0, is1):
        c = lax.axis_index("c")
        s = lax.axis_index("s")
        r0 = s * rpt
        pltpu.sync_copy(z1_hbm, zb)
        pltpu.sync_copy(zb, t_sh.at[pl.ds(r0, rpt)])
        plsc.subcore_barrier()
        base = c * eph + s * (eph // NS)
        bufs = ((sidx0, didx0, vals0, gs0, is0),
                (sidx1, didx1, vals1, gs1, is1))

        def fetch_idx(b, i):
            sidx, didx, vals, gs, isem = bufs[b]
            off = pl.multiple_of(base + i * K, K)
            pltpu.async_copy(src_hbm.at[pl.ds(off, K)], sidx, isem)
            pltpu.async_copy(dst_hbm.at[pl.ds(off, K)], didx, isem)

        def launch_gather(b, i):
            sidx, didx, vals, gs, isem = bufs[b]
            off = pl.multiple_of(base + i * K, K)
            pltpu.make_async_copy(src_hbm.at[pl.ds(off, K)], sidx, isem).wait()
            pltpu.make_async_copy(dst_hbm.at[pl.ds(off, K)], didx, isem).wait()
            pltpu.async_copy(xq_hbm.at[sidx], vals, gs)

        def step(b, i):
            sidx, didx, vals, gs, isem = bufs[b]
            pltpu.make_async_copy(xq_hbm.at[sidx], vals, gs).wait()
            pltpu.sync_copy(vals, t_sh.at[didx], add=True)

            @pl.when(i + 2 < nit)
            def _():
                fetch_idx(b, i + 2)
                launch_gather(b, i + 2)

        fetch_idx(0, 0)
        fetch_idx(1, 1)
        launch_gather(0, 0)
        launch_gather(1, 1)

        def body(i2, carry):
            step(0, 2 * i2)
            step(1, 2 * i2 + 1)
            return carry

        lax.fori_loop(0, nit // 2, body, 0)
        plsc.subcore_barrier()
        pltpu.sync_copy(t_sh.at[pl.ds(r0, rpt)], zb)
        pltpu.sync_copy(zb, t_out.at[c, pl.ds(r0, rpt)])

    return sc_scalar


def _make_sc_wide(n_pad, e_pad):
    kw = 640
    rpt = n_pad // NS
    nq = 16
    qrt = rpt // nq
    nit = e_pad // (NS * kw)
    ept = e_pad // NS

    @functools.partial(
        pl.kernel,
        out_type=jax.ShapeDtypeStruct((n_pad, 64), jnp.float32),
        mesh=_MESH,
        compiler_params=_SC_PARAMS,
        scratch_types=dict(
            sidx0=pltpu.VMEM((kw,), jnp.int32),
            didx0=pltpu.VMEM((kw,), jnp.int32),
            rows0=pltpu.VMEM((kw, 16), jnp.float32),
            sidx1=pltpu.VMEM((kw,), jnp.int32),
            didx1=pltpu.VMEM((kw,), jnp.int32),
            rows1=pltpu.VMEM((kw, 16), jnp.float32),
            bb=pltpu.VMEM((qrt, 16), jnp.float32),
            agg_sh=pltpu.VMEM_SHARED((n_pad, 16), jnp.float32),
            gs0=pltpu.SemaphoreType.DMA,
            gs1=pltpu.SemaphoreType.DMA,
            is0=pltpu.SemaphoreType.DMA,
            is1=pltpu.SemaphoreType.DMA,
        ),
    )
    def sc_wide(qv_hbm, sidx4_hbm, dst_hbm, z16_hbm, a_out,
                sidx0, didx0, rows0, sidx1, didx1, rows1, bb, agg_sh,
                gs0, gs1, is0, is1):
        c = lax.axis_index("c")
        s = lax.axis_index("s")
        r0 = s * rpt
        base = s * ept
        bufs = ((sidx0, didx0, rows0, gs0, is0),
                (sidx1, didx1, rows1, gs1, is1))
        for p in range(2):
            j = c + 2 * p
            pltpu.sync_copy(z16_hbm, bb)
            for q in range(nq):
                pltpu.sync_copy(bb, agg_sh.at[pl.ds(r0 + q * qrt, qrt), :])
            plsc.subcore_barrier()

            def fetch_idx(b, i):
                sidx, didx, rows, gs, isem = bufs[b]
                off = pl.multiple_of(base + i * kw, kw)
                joff = pl.multiple_of(j * e_pad + off, kw)
                pltpu.async_copy(sidx4_hbm.at[pl.ds(joff, kw)], sidx, isem)
                pltpu.async_copy(dst_hbm.at[pl.ds(off, kw)], didx, isem)

            def launch_gather(b, i):
                sidx, didx, rows, gs, isem = bufs[b]
                off = pl.multiple_of(base + i * kw, kw)
                joff = pl.multiple_of(j * e_pad + off, kw)
                pltpu.make_async_copy(sidx4_hbm.at[pl.ds(joff, kw)], sidx,
                                      isem).wait()
                pltpu.make_async_copy(dst_hbm.at[pl.ds(off, kw)], didx,
                                      isem).wait()
                pltpu.async_copy(qv_hbm.at[sidx], rows, gs)

            def step(b, i):
                sidx, didx, rows, gs, isem = bufs[b]
                pltpu.make_async_copy(qv_hbm.at[sidx], rows, gs).wait()
                pltpu.sync_copy(rows, agg_sh.at[didx], add=True)

                @pl.when(i + 2 < nit)
                def _():
                    fetch_idx(b, i + 2)
                    launch_gather(b, i + 2)

            fetch_idx(0, 0)
            fetch_idx(1, 1)
            launch_gather(0, 0)
            launch_gather(1, 1)

            def body(i2, carry):
                step(0, 2 * i2)
                step(1, 2 * i2 + 1)
                return carry

            lax.fori_loop(0, nit // 2, body, 0)
            plsc.subcore_barrier()
            for q in range(nq):
                rr = r0 + q * qrt
                pltpu.sync_copy(agg_sh.at[pl.ds(rr, qrt), :], bb)
                pltpu.sync_copy(bb, a_out.at[pl.ds(rr, qrt), pl.ds(j * 16, 16)])
            plsc.subcore_barrier()

    return sc_wide


def _make_tc_shift4(e_pad):
    rows = e_pad // 128
    rb = 256
    grid = rows // rb

    def body(src_ref, out_ref):
        v = src_ref[...] * 4
        out_ref[...] = v[None, :, :] + lax.broadcasted_iota(jnp.int32, (4, rb, 128), 0)

    return pl.pallas_call(
        body,
        grid=(grid,),
        in_specs=[pl.BlockSpec((rb, 128), lambda i: (i, 0))],
        out_specs=pl.BlockSpec((4, rb, 128), lambda i: (0, i, 0)),
        out_shape=jax.ShapeDtypeStruct((4, rows, 128), jnp.int32),
    )


def _make_tc_prep(n_pad):
    nblk = n_pad // BLK

    def body(degp_ref, x_ref, dinv_ref, xq_ref):
        deg = 1.0 + degp_ref[0, :] + degp_ref[1, :]
        dinv = lax.rsqrt(deg)
        dinv_ref[...] = dinv[:, None]
        xq_ref[...] = dinv[:, None] * x_ref[...]

    return pl.pallas_call(
        body,
        grid=(nblk,),
        in_specs=[
            pl.BlockSpec((2, BLK), lambda i: (0, i)),
            pl.BlockSpec((BLK, 1), lambda i: (i, 0)),
        ],
        out_specs=[
            pl.BlockSpec((BLK, 1), lambda i: (i, 0)),
            pl.BlockSpec((BLK, 1), lambda i: (i, 0)),
        ],
        out_shape=[
            jax.ShapeDtypeStruct((n_pad, 1), jnp.float32),
            jax.ShapeDtypeStruct((n_pad, 1), jnp.float32),
        ],
    )


def _make_tc_layer1(n_pad, d):
    nblk = n_pad // BLK

    def body(tp_ref, xq_ref, dinv_ref, w0_ref, b0_ref, w1_ref, q_ref):
        xq = xq_ref[...]
        dinv = dinv_ref[...]
        u = dinv * (xq + tp_ref[0, :][:, None] + tp_ref[1, :][:, None])
        h = _elu(u * w0_ref[...] + b0_ref[...])
        q_ref[...] = dinv * jnp.dot(h, w1_ref[...],
                                    preferred_element_type=jnp.float32)

    return pl.pallas_call(
        body,
        grid=(nblk,),
        in_specs=[
            pl.BlockSpec((2, BLK), lambda i: (0, i)),
            pl.BlockSpec((BLK, 1), lambda i: (i, 0)),
            pl.BlockSpec((BLK, 1), lambda i: (i, 0)),
            pl.BlockSpec((1, d), lambda i: (0, 0)),
            pl.BlockSpec((1, d), lambda i: (0, 0)),
            pl.BlockSpec((d, d), lambda i: (0, 0)),
        ],
        out_specs=pl.BlockSpec((BLK, d), lambda i: (i, 0)),
        out_shape=jax.ShapeDtypeStruct((n_pad, d), jnp.float32),
    )


def _make_tc_layer(n_pad, d):
    nblk = n_pad // BLK

    def body(a_ref, q_ref, dinv_ref, b_ref, w_ref, qn_ref):
        dinv = dinv_ref[...]
        h = _elu(dinv * (a_ref[...] + q_ref[...]) + b_ref[...])
        qn_ref[...] = dinv * jnp.dot(h, w_ref[...],
                                     preferred_element_type=jnp.float32)

    return pl.pallas_call(
        body,
        grid=(nblk,),
        in_specs=[
            pl.BlockSpec((BLK, d), lambda i: (i, 0)),
            pl.BlockSpec((BLK, d), lambda i: (i, 0)),
            pl.BlockSpec((BLK, 1), lambda i: (i, 0)),
            pl.BlockSpec((1, d), lambda i: (0, 0)),
            pl.BlockSpec((d, d), lambda i: (0, 0)),
        ],
        out_specs=pl.BlockSpec((BLK, d), lambda i: (i, 0)),
        out_shape=jax.ShapeDtypeStruct((n_pad, d), jnp.float32),
    )


def _make_tc_final(n_pad, d):
    nblk = n_pad // BLK

    def body(a_ref, q_ref, dinv_ref, b_ref, batch_ref, lw_ref, lb_ref,
             out_ref, sums, cnt):
        i = pl.program_id(0)
        dinv = dinv_ref[...]
        h = _elu(dinv * (a_ref[...] + q_ref[...]) + b_ref[...])
        gid = lax.broadcasted_iota(jnp.int32, (BLK, NG), 1)
        oh = (batch_ref[...] == gid).astype(jnp.float32)
        psum = lax.dot_general(oh, h, (((0,), (0,)), ((), ())),
                               preferred_element_type=jnp.float32)
        pcnt = jnp.sum(oh, axis=0)[:, None]

        @pl.when(i == 0)
        def _():
            sums[...] = psum
            cnt[...] = pcnt

        @pl.when(i > 0)
        def _():
            sums[...] += psum
            cnt[...] += pcnt

        @pl.when(i == nblk - 1)
        def _():
            pooled = sums[...] / jnp.maximum(cnt[...], 1.0)
            out_ref[...] = jnp.dot(pooled, lw_ref[...],
                                   preferred_element_type=jnp.float32) + lb_ref[...]

    return pl.pallas_call(
        body,
        grid=(nblk,),
        in_specs=[
            pl.BlockSpec((BLK, d), lambda i: (i, 0)),
            pl.BlockSpec((BLK, d), lambda i: (i, 0)),
            pl.BlockSpec((BLK, 1), lambda i: (i, 0)),
            pl.BlockSpec((1, d), lambda i: (0, 0)),
            pl.BlockSpec((BLK, 1), lambda i: (i, 0)),
            pl.BlockSpec((d, 1), lambda i: (0, 0)),
            pl.BlockSpec((1, 1), lambda i: (0, 0)),
        ],
        out_specs=pl.BlockSpec((NG, 1), lambda i: (0, 0)),
        out_shape=jax.ShapeDtypeStruct((NG, 1), jnp.float32),
        scratch_shapes=[
            pltpu.VMEM((NG, d), jnp.float32),
            pltpu.VMEM((NG, 1), jnp.float32),
        ],
    )


def kernel(x, edge_index, batch, W0, b0, W_rest, b_rest, lin_W, lin_b):
    n = x.shape[0]
    e = edge_index.shape[1]
    d = W0.shape[1]
    nl = W_rest.shape[0] + 1

    n_pad = _ceil_to(n + 1, BLK)
    e_pad = _ceil_to(e, 327680)
    dummy = n

    src = edge_index[0].astype(jnp.int32)
    dst = edge_index[1].astype(jnp.int32)
    src_p = jnp.pad(src, (0, e_pad - e))
    dst_p = jnp.pad(dst, (0, e_pad - e), constant_values=dummy)
    x_p = jnp.pad(x.astype(jnp.float32), ((0, n_pad - n), (0, 0)))
    batch_p = jnp.pad(batch.astype(jnp.int32), (0, n_pad - n),
                      constant_values=-1).reshape(n_pad, 1)

    ones_k = jnp.ones((K,), jnp.float32)
    z1 = jnp.zeros((n_pad // NS,), jnp.float32)
    z16 = jnp.zeros((n_pad // NS // 16, 16), jnp.float32)

    sidx4 = _make_tc_shift4(e_pad)(src_p.reshape(e_pad // 128, 128))
    sidx4_flat = sidx4.reshape(4 * e_pad)

    deg_p = _make_sc_deg(n_pad, e_pad)(dst_p, ones_k, z1)
    dinv, xq = _make_tc_prep(n_pad)(deg_p, x_p)

    t_p = _make_sc_scalar(n_pad, e_pad)(xq.reshape(n_pad), src_p, dst_p, z1)
    q = _make_tc_layer1(n_pad, d)(t_p, xq, dinv, W0, b0.reshape(1, d),
                                  W_rest[0])

    sc_wide = _make_sc_wide(n_pad, e_pad)
    tc_layer = _make_tc_layer(n_pad, d)
    for l in range(nl - 2):
        a = sc_wide(q.reshape(4 * n_pad, 16), sidx4_flat, dst_p, z16)
        q = tc_layer(a, q, dinv, b_rest[l].reshape(1, d), W_rest[l + 1])

    a = sc_wide(q.reshape(4 * n_pad, 16), sidx4_flat, dst_p, z16)
    out = _make_tc_final(n_pad, d)(a, q, dinv, b_rest[nl - 2].reshape(1, d),
                                   batch_p, lin_W, lin_b.reshape(1, 1))
    return out.reshape(NG)
